# baseline (speedup 1.0000x reference)
import functools

import jax
import jax.numpy as jnp
from jax import lax
from jax.experimental import pallas as pl

S, D, C, K = 2048, 768, 24576, 32
EBLK = 1024
NEB = C // EBLK
DBLK = 512
NDB = C // DBLK


def _encode_body(a_ref, w_ref, b_ref, o_ref):
    a = a_ref[...]
    w = w_ref[...]
    acc = lax.dot_general(a, w, (((1,), (1,)), ((), ())),
                          preferred_element_type=jnp.float32)
    o_ref[...] = acc + b_ref[0:1, :]


def _encode(act2d, W_enc, b_enc2d):
    return pl.pallas_call(
        _encode_body,
        grid=(NEB,),
        in_specs=[
            pl.BlockSpec((S, D), lambda i: (0, 0)),
            pl.BlockSpec((EBLK, D), lambda i: (i, 0)),
            pl.BlockSpec((8, EBLK), lambda i: (0, i)),
        ],
        out_specs=pl.BlockSpec((S, EBLK), lambda i: (0, i)),
        out_shape=jax.ShapeDtypeStruct((S, C), jnp.float32),
    )(act2d, W_enc, b_enc2d)


def _decode_body(p_ref, w_ref, t_ref, o_ref):
    i = pl.program_id(0)
    p = p_ref[...]
    t = t_ref[...]
    masked = jnp.where(p >= t, p, 0.0)
    acc = lax.dot_general(masked, w_ref[...], (((1,), (1,)), ((), ())),
                          preferred_element_type=jnp.float32)

    @pl.when(i == 0)
    def _():
        o_ref[...] = acc

    @pl.when(i != 0)
    def _():
        o_ref[...] += acc


def _decode(pre_act, W_emb, thr):
    return pl.pallas_call(
        _decode_body,
        grid=(NDB,),
        in_specs=[
            pl.BlockSpec((S, DBLK), lambda i: (0, i)),
            pl.BlockSpec((D, DBLK), lambda i: (0, i)),
            pl.BlockSpec((S, 1), lambda i: (0, 0)),
        ],
        out_specs=pl.BlockSpec((S, D), lambda i: (0, 0)),
        out_shape=jax.ShapeDtypeStruct((S, D), jnp.float32),
    )(pre_act, W_emb, thr)


def kernel(activations, W_enc, b_enc, W_emb):
    B = activations.shape[0]
    act2d = activations.reshape(B * S, D)
    b2d = jnp.broadcast_to(b_enc.reshape(1, C), (8, C))
    pre_act = _encode(act2d, W_enc, b2d)
    top_vals = lax.top_k(pre_act, K)[0]
    thr = top_vals[:, K - 1:K]
    out = _decode(pre_act, W_emb, thr)
    return out.reshape(B, S, D)


# speedup vs baseline: 14.1851x; 14.1851x over previous
import functools

import jax
import jax.numpy as jnp
from jax import lax
from jax.experimental import pallas as pl
from jax.experimental.pallas import tpu as pltpu
from jax.experimental.pallas import tpu_sc as plsc

S, D, C, K = 2048, 768, 24576, 32
EBLK = 1024
NEB = C // EBLK
DBLK = 512
NDB = C // DBLK
CW = 128
NCHUNK = C // CW
NC, NS, L = 2, 16, 16
NW = NC * NS
RPW = S // NW


def _encode_body(a_ref, w_ref, b_ref, o_ref, m_ref):
    a = a_ref[...]
    w = w_ref[...]
    acc = lax.dot_general(a, w, (((1,), (1,)), ((), ())),
                          preferred_element_type=jnp.float32)
    acc = acc + b_ref[0:1, :]
    o_ref[...] = acc
    cols = [jnp.max(acc[:, j * CW:(j + 1) * CW], axis=-1, keepdims=True)
            for j in range(EBLK // CW)]
    pad = jnp.full((S, CW - EBLK // CW), -jnp.inf, jnp.float32)
    m_ref[...] = jnp.concatenate(cols + [pad], axis=-1)


def _encode(act2d, W_enc, b_enc2d):
    return pl.pallas_call(
        _encode_body,
        grid=(NEB,),
        in_specs=[
            pl.BlockSpec((S, D), lambda i: (0, 0)),
            pl.BlockSpec((EBLK, D), lambda i: (i, 0)),
            pl.BlockSpec((8, EBLK), lambda i: (0, i)),
        ],
        out_specs=[
            pl.BlockSpec((S, EBLK), lambda i: (0, i)),
            pl.BlockSpec((S, CW), lambda i: (0, i)),
        ],
        out_shape=[
            jax.ShapeDtypeStruct((S, C), jnp.float32),
            jax.ShapeDtypeStruct((S, NEB * CW), jnp.float32),
        ],
    )(act2d, W_enc, b_enc2d)


def _decode_body(p_ref, w_ref, t_ref, o_ref):
    i = pl.program_id(0)
    p = p_ref[...]
    t = t_ref[...]
    masked = jnp.where(p >= t, p, 0.0)
    acc = lax.dot_general(masked, w_ref[...], (((1,), (1,)), ((), ())),
                          preferred_element_type=jnp.float32)

    @pl.when(i == 0)
    def _():
        o_ref[...] = acc

    @pl.when(i != 0)
    def _():
        o_ref[...] += acc


def _decode(pre_act, W_emb, thr):
    return pl.pallas_call(
        _decode_body,
        grid=(NDB,),
        in_specs=[
            pl.BlockSpec((S, DBLK), lambda i: (0, i)),
            pl.BlockSpec((D, DBLK), lambda i: (0, i)),
            pl.BlockSpec((S, 1), lambda i: (0, 0)),
        ],
        out_specs=pl.BlockSpec((S, D), lambda i: (0, 0)),
        out_shape=jax.ShapeDtypeStruct((S, D), jnp.float32),
    )(pre_act, W_emb, thr)


import numpy as np

_MSB = np.int32(-2147483648)


def _iota16():
    return lax.iota(jnp.int32, L)


def _monotone(xi):
    return jnp.where(xi >= 0, xi | _MSB, ~xi)


def _inv_monotone(u):
    return jnp.where(u < 0, u ^ _MSB, ~u)


def _uge(a, b):
    return (a ^ _MSB) >= (b ^ _MSB)


def _digit(u, lvl):
    return lax.shift_right_logical(u, jnp.int32(28 - 4 * lvl)) & jnp.int32(15)


def _radix_select(load_fn, n, rank, hist, bufs):
    thresh = jnp.zeros((L,), jnp.int32)
    rank = jnp.int32(rank)
    n = jnp.int32(n)
    src = load_fn
    for lvl in range(8):
        for j in range(16):
            hist[pl.ds(j * L, L)] = jnp.zeros((L,), jnp.int32)
        nv = lax.div(n + 15, jnp.int32(16))

        def hist_body(i, carry, src=src, lvl=lvl, n=n):
            u = src(i)
            digit = _digit(u, lvl)
            valid = (_iota16() + i * L) < n
            plsc.addupdate_scatter(hist, [_iota16() * 16 + digit],
                                   jnp.ones((L,), jnp.int32), mask=valid)
            return carry

        lax.fori_loop(0, nv, hist_body, jnp.int32(0))
        htot = hist[pl.ds(0, L)]
        for j in range(1, 16):
            htot = htot + hist[pl.ds(j * L, L)]
        scum = plsc.cumsum(lax.rev(htot, (0,)))
        k = jnp.max(plsc.all_reduce_ffs(scum >= rank))
        b = 15 - k
        prev = jnp.sum(jnp.where(_iota16() == (k - 1), scum, 0))
        rank = rank - prev
        bvec = jnp.broadcast_to(b, (L,)).astype(jnp.int32)
        thresh = thresh | lax.shift_left(bvec, jnp.int32(28 - 4 * lvl))
        if lvl < 7:
            dst = bufs[lvl % 2]

            def comp_body(i, w, src=src, lvl=lvl, n=n, bvec=bvec, dst=dst):
                u = src(i)
                digit = _digit(u, lvl)
                valid = (_iota16() + i * L) < n
                keep = (digit == bvec) & valid
                plsc.store_compressed(dst.at[pl.ds(w, L)], u, mask=keep)
                return w + jnp.max(plsc.all_reduce_population_count(keep))

            n = lax.fori_loop(0, nv, comp_body, jnp.int32(0))
            src = (lambda i, dst=dst: dst[pl.ds(i * L, L)])
    return thresh


def _sc_body(cmax_hbm, pre2d_hbm, out_hbm,
             cmax_v, cmd_v, hist_v, sel_v, cand_v, ping_v, pong_v, out_v, sem):
    cid = lax.axis_index("c")
    sid = lax.axis_index("s")
    wid = sid * NC + cid
    low8 = _iota16() < 8

    def row_body(r, carry):
        row = wid * RPW + r
        pltpu.sync_copy(cmax_hbm.at[row], cmax_v)

        for jb in range(NEB):
            v = _monotone(cmax_v[pl.ds(jb * CW, L)])
            plsc.store_compressed(cmd_v.at[pl.ds(jb * 8, L)], v, mask=low8)

        def cm_load(i):
            return cmd_v[pl.ds(i * L, L)]

        t_u = _radix_select(cm_load, NCHUNK, K, hist_v, (ping_v, pong_v))

        rowbase = row * NCHUNK
        fill = jnp.broadcast_to(rowbase, (L,)).astype(jnp.int32)
        for j in range(NCHUNK // L):
            sel_v[pl.ds(j * L, L)] = fill

        def sel_body(j, w):
            u = cmd_v[pl.ds(j * L, L)]
            keep = _uge(u, t_u)
            ids = rowbase + j * L + _iota16()
            plsc.store_compressed(sel_v.at[pl.ds(w, L)], ids, mask=keep)
            return w + jnp.max(plsc.all_reduce_population_count(keep))

        m = lax.fori_loop(0, NCHUNK // L, sel_body, jnp.int32(0))

        g = lax.div(m + 47, jnp.int32(48))

        def gat_body(j, carry):
            pltpu.async_copy(
                pre2d_hbm.at[sel_v.at[pl.ds(j * 48, 48)]],
                cand_v.at[pl.ds(j * 48, 48)], sem).wait()
            return carry

        lax.fori_loop(0, g, gat_body, jnp.int32(0))

        def cand_load(i):
            chunk = i >> 3
            off = (i & 7) * L
            return _monotone(cand_v[chunk, pl.ds(off, L)])

        v_u = _radix_select(cand_load, m * CW, K, hist_v, (ping_v, pong_v))
        thr_bits = _inv_monotone(v_u)
        plsc.store_scatter(out_v, [jnp.broadcast_to(r, (L,)).astype(jnp.int32)],
                           thr_bits, mask=_iota16() == 0)
        return carry

    lax.fori_loop(0, RPW, row_body, jnp.int32(0))
    pltpu.sync_copy(out_v, out_hbm.at[pl.ds(wid * RPW, RPW)])


@functools.partial(
    pl.kernel,
    out_type=jax.ShapeDtypeStruct((S,), jnp.int32),
    mesh=plsc.VectorSubcoreMesh(core_axis_name="c", subcore_axis_name="s"),
    compiler_params=pltpu.CompilerParams(needs_layout_passes=False),
    scratch_types=[
        pltpu.VMEM((NEB * CW,), jnp.int32),
        pltpu.VMEM((NCHUNK + L,), jnp.int32),
        pltpu.VMEM((256,), jnp.int32),
        pltpu.VMEM((NCHUNK,), jnp.int32),
        pltpu.VMEM((NCHUNK, CW), jnp.int32),
        pltpu.VMEM((C + L,), jnp.int32),
        pltpu.VMEM((C + L,), jnp.int32),
        pltpu.VMEM((RPW,), jnp.int32),
        pltpu.SemaphoreType.DMA,
    ],
)
def _sc_thresh(cmax_hbm, pre2d_hbm, out_hbm,
               cmax_v, cmd_v, hist_v, sel_v, cand_v, ping_v, pong_v, out_v,
               sem):
    _sc_body(cmax_hbm, pre2d_hbm, out_hbm,
             cmax_v, cmd_v, hist_v, sel_v, cand_v, ping_v, pong_v, out_v, sem)


def kernel(activations, W_enc, b_enc, W_emb):
    B = activations.shape[0]
    act2d = activations.reshape(B * S, D)
    b2d = jnp.broadcast_to(b_enc.reshape(1, C), (8, C))
    pre_act, cmax = _encode(act2d, W_enc, b2d)
    cmax_bits = lax.bitcast_convert_type(cmax, jnp.int32)
    pre_bits = lax.bitcast_convert_type(pre_act, jnp.int32).reshape(
        S * NCHUNK, CW)
    thr_bits = _sc_thresh(cmax_bits, pre_bits)
    thr = lax.bitcast_convert_type(thr_bits, jnp.float32)
    out = _decode(pre_act, W_emb, thr.reshape(S, 1))
    return out.reshape(B, S, D)
